# baseline (speedup 1.0000x reference)
import functools

import jax
import jax.numpy as jnp
import numpy as np
from jax import lax
from jax.experimental import pallas as pl

_H = jax.lax.Precision.HIGHEST

N = 192
NOPS = 64
WIDTH = 8
ROWS = 24

_P = np.zeros((N, 16), dtype=np.float32)
for _y in range(ROWS):
    for _x in range(WIDTH):
        _P[_y * WIDTH + _x, (_y // 6) * 4 + (_x // 2)] = 1.0


def _bf(v):
    return v.astype(jnp.bfloat16).astype(jnp.float32)


def _colb(v_row, n):
    ones = jnp.ones((1, n), jnp.float32)
    return lax.dot_general(v_row, ones, (((0,), (0,)), ((), ())), precision=_H)


def _stable_rank(v_row, n):
    vcol = _colb(v_row, n)
    vrow = jnp.broadcast_to(v_row, (n, n))
    jj = lax.broadcasted_iota(jnp.int32, (n, n), 0)
    ii = lax.broadcasted_iota(jnp.int32, (n, n), 1)
    cmp = (vcol < vrow) | ((vcol == vrow) & (jj < ii))
    return jnp.sum(cmp.astype(jnp.float32), axis=0)


def _perm_rows(vals_row, rank_row, n):
    rcol = _colb(rank_row, n)
    vcol = _colb(vals_row, n)
    sel = (rcol == lax.broadcasted_iota(jnp.int32, (n, n), 1).astype(jnp.float32))
    return jnp.sum(jnp.where(sel, vcol, 0.0), axis=0)


def _staged(h):
    mult = jnp.where(h <= 2.0, 1.0,
           jnp.where(h <= 4.0, 1.5,
           jnp.where(h <= 8.0, 2.0,
           jnp.where(h <= 16.0, 3.0, 5.0))))
    return h * mult


def _penalty(x):
    f = jnp.maximum(x, 0.0)
    b = jnp.maximum(-x, 0.0)
    fm = _staged(f)
    bm = _staged(b) * b
    return jnp.sum(fm + bm)


def _body(x_ref, cw_ref, cb_ref, mw_ref, mb_ref, pw_ref, pb_ref, pool_ref,
          inter_ref, intra_ref):
    x = x_ref[0]
    rank = _stable_rank(x, N)
    rank_row = rank[None, :]

    idx_row = lax.broadcasted_iota(jnp.int32, (1, N), 1).astype(jnp.float32)
    sortf = _perm_rows(idx_row, rank_row, N)

    feats_parts = []
    for m in range(3):
        a = sortf[m * 64:(m + 1) * 64][None, :]
        ap = jnp.pad(a, ((0, 0), (16, 16)))
        xi = lax.broadcasted_iota(jnp.int32, (1, 64), 1) % 8
        acc = jnp.broadcast_to(cb_ref[m], (8, 64))
        for dy in range(3):
            for dx in range(3):
                delta = 8 * (dy - 1) + (dx - 1)
                sh = ap[:, 16 + delta:80 + delta]
                if dx == 0:
                    sh = jnp.where(xi == 0, 0.0, sh)
                elif dx == 2:
                    sh = jnp.where(xi == 7, 0.0, sh)
                acc = acc + _bf(cw_ref[m, dy * 3 + dx]) * sh
        feats_parts.append(jnp.maximum(acc, 0.0))
    feats = jnp.concatenate(feats_parts, axis=1)

    o2 = (lax.broadcasted_iota(jnp.int32, (N, N), 0).astype(jnp.float32) ==
          jnp.broadcast_to(rank_row, (N, N))).astype(jnp.float32)
    mem = lax.dot_general(feats, o2, (((1,), (0,)), ((), ())), precision=_H)

    memp = jnp.pad(_bf(mem), ((0, 0), (16, 16)))
    xi2 = lax.broadcasted_iota(jnp.int32, (1, N), 1) % 8
    acc2 = jnp.broadcast_to(mb_ref[...], (16, N))
    for dy in range(3):
        for dx in range(3):
            delta = 8 * (dy - 1) + (dx - 1)
            sh = memp[:, 16 + delta:208 + delta]
            if dx == 0:
                sh = jnp.where(xi2 == 0, 0.0, sh)
            elif dx == 2:
                sh = jnp.where(xi2 == 7, 0.0, sh)
            for c in range(8):
                acc2 = acc2 + _bf(mw_ref[dy * 3 + dx, c]) * sh[c:c + 1, :]
    mc = jnp.maximum(acc2, 0.0)

    pooled = lax.dot_general(mc, pool_ref[...], (((1,), (0,)), ((), ())),
                             precision=_H) * (1.0 / 12.0)
    t = _bf(pw_ref[...]) * _bf(pooled)[None, :, :]
    ol = jnp.sum(jnp.sum(t, axis=2), axis=1)[None, :] + pb_ref[...]

    orank = _stable_rank(ol, NOPS)[None, :]
    s0 = _perm_rows(sortf[0:64][None, :], orank, NOPS)
    s1 = _perm_rows(sortf[64:128][None, :], orank, NOPS)
    d = _perm_rows(sortf[128:192][None, :], orank, NOPS)

    intra = jnp.concatenate([s1 - s0, d - s1])
    inter = s0[1:] - d[:-1]
    inter_ref[0, 0, :] = jnp.zeros((128,), jnp.float32) + _penalty(inter)
    intra_ref[0, 0, :] = jnp.zeros((128,), jnp.float32) + _penalty(intra)


@jax.jit
def kernel(mem_logits_batch, conv_w, conv_b, mem_conv_w, mem_conv_b,
           proj_w, proj_b):
    B = mem_logits_batch.shape[0]
    cw_t = jnp.transpose(conv_w[:, :, 0], (0, 2, 3, 1)).reshape(3, 9, 8, 1)
    cb_t = conv_b.reshape(3, 8, 1)
    mw_t = jnp.transpose(mem_conv_w, (2, 3, 1, 0)).reshape(9, 8, 16, 1)
    mb_t = mem_conv_b.reshape(16, 1)
    pw3 = proj_w.reshape(64, 16, 16)
    pb2 = proj_b.reshape(1, 64)
    pool = jnp.asarray(_P)

    specs = [
        pl.BlockSpec((1, 1, N), lambda i: (i, 0, 0)),
        pl.BlockSpec((3, 9, 8, 1), lambda i: (0, 0, 0, 0)),
        pl.BlockSpec((3, 8, 1), lambda i: (0, 0, 0)),
        pl.BlockSpec((9, 8, 16, 1), lambda i: (0, 0, 0, 0)),
        pl.BlockSpec((16, 1), lambda i: (0, 0)),
        pl.BlockSpec((64, 16, 16), lambda i: (0, 0, 0)),
        pl.BlockSpec((1, 64), lambda i: (0, 0)),
        pl.BlockSpec((N, 16), lambda i: (0, 0)),
    ]
    out_specs = [pl.BlockSpec((1, 1, 128), lambda i: (i, 0, 0)),
                 pl.BlockSpec((1, 1, 128), lambda i: (i, 0, 0))]
    out_shape = [jax.ShapeDtypeStruct((B, 1, 128), jnp.float32),
                 jax.ShapeDtypeStruct((B, 1, 128), jnp.float32)]
    inter, intra = pl.pallas_call(
        _body,
        grid=(B,),
        in_specs=specs,
        out_specs=out_specs,
        out_shape=out_shape,
    )(mem_logits_batch.reshape(B, 1, N), cw_t, cb_t, mw_t, mb_t, pw3, pb2,
      pool)
    return inter[:, 0, 0], intra[:, 0, 0]


# speedup vs baseline: 1.3110x; 1.3110x over previous
import jax
import jax.numpy as jnp
import numpy as np
from jax import lax
from jax.experimental import pallas as pl

_H = jax.lax.Precision.HIGHEST
_BF = jnp.bfloat16
_F = jnp.float32

N = 192
NOPS = 64
WIDTH = 8
ROWS = 24

_P = np.zeros((N, 16), dtype=np.float32)
for _y in range(ROWS):
    for _x in range(WIDTH):
        _P[_y * WIDTH + _x, (_y // 6) * 4 + (_x // 2)] = 1.0


def _bf(v):
    return v.astype(_BF).astype(_F)


def _dotbf(a, b):
    return lax.dot_general(a.astype(_BF), b.astype(_BF),
                           (((1,), (0,)), ((), ())),
                           preferred_element_type=_F)


def _colb3(v_row, n):
    hi = _bf(v_row)
    lo = _bf(v_row - hi)
    rest = v_row - hi - lo
    ones = jnp.ones((1, n), _BF)

    def outer(part):
        return lax.dot_general(part.astype(_BF), ones,
                               (((0,), (0,)), ((), ())),
                               preferred_element_type=_F)
    return outer(hi) + outer(lo) + outer(rest)


def _colb1(v_row, n):
    ones = jnp.ones((1, n), _BF)
    return lax.dot_general(v_row.astype(_BF), ones,
                           (((0,), (0,)), ((), ())),
                           preferred_element_type=_F)


def _cmp_matrix(v_row, n):
    vcol = _colb3(v_row, n)
    vrow = jnp.broadcast_to(v_row, (n, n))
    jj = lax.broadcasted_iota(jnp.int32, (n, n), 0)
    ii = lax.broadcasted_iota(jnp.int32, (n, n), 1)
    cmp = (vcol < vrow) | ((vcol == vrow) & (jj < ii))
    return cmp.astype(_F)


def _staged(h):
    return jnp.where(h <= 2.0, 1.0,
           jnp.where(h <= 4.0, 1.5,
           jnp.where(h <= 8.0, 2.0,
           jnp.where(h <= 16.0, 3.0, 5.0))))


def _penalty(x):
    f = jnp.maximum(x, 0.0)
    b = jnp.maximum(-x, 0.0)
    return jnp.sum(f * _staged(f) + b * b * _staged(b))


def _body(x_ref, w1_ref, cb_ref, w2_ref, mb_ref, pwt_ref, pb_ref, pool_ref,
          inter_ref, intra_ref):
    x = x_ref[0]
    cmp = _cmp_matrix(x, N)
    rank_row = jnp.sum(cmp, axis=0)[None, :]
    rank_col = (float(N - 1) -
                jnp.sum(cmp, axis=1))[:, None]

    rr = lax.broadcasted_iota(jnp.int32, (N, N), 1).astype(_F)
    pp = lax.broadcasted_iota(jnp.int32, (N, N), 0).astype(_F)
    sel = (jnp.broadcast_to(rank_col, (N, N)) == rr)
    sortf = jnp.sum(jnp.where(sel, pp, 0.0), axis=0)[None, :]

    xi = lax.broadcasted_iota(jnp.int32, (1, 64), 1) % 8
    rows = []
    for m in range(3):
        a = sortf[:, m * 64:(m + 1) * 64]
        ap = jnp.pad(a, ((0, 0), (16, 16)))
        for dy in range(3):
            for dx in range(3):
                delta = 8 * (dy - 1) + (dx - 1)
                sh = ap[:, 16 + delta:80 + delta]
                if dx == 0:
                    sh = jnp.where(xi == 0, 0.0, sh)
                elif dx == 2:
                    sh = jnp.where(xi == 7, 0.0, sh)
                rows.append(sh)
    patches = jnp.concatenate(rows, axis=0)
    f24 = jnp.maximum(_dotbf(w1_ref[...], patches) + cb_ref[...], 0.0)
    feats = jnp.concatenate([f24[8 * m:8 * m + 8] for m in range(3)],
                            axis=1)

    o2 = (jnp.broadcast_to(rank_row, (N, N)) ==
          lax.broadcasted_iota(jnp.int32, (N, N), 0).astype(_F))
    mem = _dotbf(feats, o2.astype(_F))

    memp = jnp.pad(mem, ((0, 0), (16, 16)))
    xi2 = lax.broadcasted_iota(jnp.int32, (1, N), 1) % 8
    rows2 = []
    for dy in range(3):
        for dx in range(3):
            delta = 8 * (dy - 1) + (dx - 1)
            sh = memp[:, 16 + delta:208 + delta]
            if dx == 0:
                sh = jnp.where(xi2 == 0, 0.0, sh)
            elif dx == 2:
                sh = jnp.where(xi2 == 7, 0.0, sh)
            rows2.append(sh)
    patches2 = jnp.concatenate(rows2, axis=0)
    mc = jnp.maximum(_dotbf(w2_ref[...], patches2) + mb_ref[...], 0.0)

    pooled = lax.dot_general(mc, pool_ref[...], (((1,), (0,)), ((), ())),
                             precision=_H) * (1.0 / 12.0)
    t = _bf(pwt_ref[...]) * _bf(pooled)[None, :, :]
    ol = jnp.sum(jnp.sum(t, axis=2), axis=1)[None, :] + pb_ref[...]

    cmp2 = _cmp_matrix(ol, NOPS)
    orank_col = (float(NOPS - 1) - jnp.sum(cmp2, axis=1))[:, None]
    rr2 = lax.broadcasted_iota(jnp.int32, (NOPS, NOPS), 1).astype(_F)
    sel2 = (jnp.broadcast_to(orank_col, (NOPS, NOPS)) == rr2)

    def permute(vals_row):
        vcol = _colb1(vals_row, NOPS)
        return jnp.sum(jnp.where(sel2, vcol, 0.0), axis=0)

    s0 = permute(sortf[:, 0:64])
    s1 = permute(sortf[:, 64:128])
    d = permute(sortf[:, 128:192])

    intra = jnp.concatenate([s1 - s0, d - s1])
    inter = s0[1:] - d[:-1]
    inter_ref[0, 0, :] = jnp.zeros((128,), _F) + _penalty(inter)
    intra_ref[0, 0, :] = jnp.zeros((128,), _F) + _penalty(intra)


@jax.jit
def kernel(mem_logits_batch, conv_w, conv_b, mem_conv_w, mem_conv_b,
           proj_w, proj_b):
    B = mem_logits_batch.shape[0]
    w1 = jnp.zeros((24, 27), _F)
    cw = conv_w[:, :, 0]
    for m in range(3):
        w1 = w1.at[8 * m:8 * m + 8, 9 * m:9 * m + 9].set(
            cw[m].reshape(8, 9))
    cb24 = conv_b.reshape(24, 1)
    w2 = jnp.transpose(mem_conv_w, (2, 3, 1, 0)).reshape(72, 16).T
    mb_t = mem_conv_b.reshape(16, 1)
    pwt = proj_w.reshape(64, 16, 16)
    pb2 = proj_b.reshape(1, 64)
    pool = jnp.asarray(_P)

    specs = [
        pl.BlockSpec((1, 1, N), lambda i: (i, 0, 0)),
        pl.BlockSpec((24, 27), lambda i: (0, 0)),
        pl.BlockSpec((24, 1), lambda i: (0, 0)),
        pl.BlockSpec((16, 72), lambda i: (0, 0)),
        pl.BlockSpec((16, 1), lambda i: (0, 0)),
        pl.BlockSpec((64, 16, 16), lambda i: (0, 0, 0)),
        pl.BlockSpec((1, 64), lambda i: (0, 0)),
        pl.BlockSpec((N, 16), lambda i: (0, 0)),
    ]
    out_specs = [pl.BlockSpec((1, 1, 128), lambda i: (i, 0, 0)),
                 pl.BlockSpec((1, 1, 128), lambda i: (i, 0, 0))]
    out_shape = [jax.ShapeDtypeStruct((B, 1, 128), _F),
                 jax.ShapeDtypeStruct((B, 1, 128), _F)]
    inter, intra = pl.pallas_call(
        _body,
        grid=(B,),
        in_specs=specs,
        out_specs=out_specs,
        out_shape=out_shape,
    )(mem_logits_batch.reshape(B, 1, N), w1, cb24, w2, mb_t, pwt, pb2, pool)
    return inter[:, 0, 0], intra[:, 0, 0]


# speedup vs baseline: 1.7385x; 1.3261x over previous
import jax
import jax.numpy as jnp
import numpy as np
from jax import lax
from jax.experimental import pallas as pl

_H = jax.lax.Precision.HIGHEST
_BF = jnp.bfloat16
_F = jnp.float32

N = 192
NOPS = 64
WIDTH = 8
ROWS = 24

_P = np.zeros((N, 16), dtype=np.float32)
for _y in range(ROWS):
    for _x in range(WIDTH):
        _P[_y * WIDTH + _x, (_y // 6) * 4 + (_x // 2)] = 1.0


def _bf(v):
    return v.astype(_BF).astype(_F)


def _dotbf(a, b):
    return lax.dot_general(a.astype(_BF), b.astype(_BF),
                           (((1,), (0,)), ((), ())),
                           preferred_element_type=_F)


def _colb3(v_row, n):
    hi = _bf(v_row)
    lo = _bf(v_row - hi)
    rest = v_row - hi - lo
    ones = jnp.ones((1, n), _BF)

    def outer(part):
        return lax.dot_general(part.astype(_BF), ones,
                               (((0,), (0,)), ((), ())),
                               preferred_element_type=_F)
    return outer(hi) + outer(lo) + outer(rest)


def _colb1(v_row, n):
    ones = jnp.ones((1, n), _BF)
    return lax.dot_general(v_row.astype(_BF), ones,
                           (((0,), (0,)), ((), ())),
                           preferred_element_type=_F)


def _cmp_matrix(v_row, n):
    vcol = _colb3(v_row, n)
    vrow = jnp.broadcast_to(v_row, (n, n))
    jj = lax.broadcasted_iota(jnp.int32, (n, n), 0)
    ii = lax.broadcasted_iota(jnp.int32, (n, n), 1)
    cmp = (vcol < vrow) | ((vcol == vrow) & (jj < ii))
    return cmp.astype(_F)


def _staged(h):
    return jnp.where(h <= 2.0, 1.0,
           jnp.where(h <= 4.0, 1.5,
           jnp.where(h <= 8.0, 2.0,
           jnp.where(h <= 16.0, 3.0, 5.0))))


def _penalty(x):
    f = jnp.maximum(x, 0.0)
    b = jnp.maximum(-x, 0.0)
    return jnp.sum(f * _staged(f) + b * b * _staged(b))


SPB = 4


def _body(x_ref, w1_ref, cb_ref, w2_ref, mb_ref, pwt_ref, pb_ref, pool_ref,
          inter_ref, intra_ref):
    for s in range(SPB):
        ip, xp = _sample(x_ref[s], w1_ref, cb_ref, w2_ref, mb_ref, pwt_ref,
                         pb_ref, pool_ref)
        inter_ref[s, 0, :] = jnp.zeros((128,), _F) + ip
        intra_ref[s, 0, :] = jnp.zeros((128,), _F) + xp


def _sample(x, w1_ref, cb_ref, w2_ref, mb_ref, pwt_ref, pb_ref, pool_ref):
    cmp = _cmp_matrix(x, N)
    rank_row = jnp.sum(cmp, axis=0)[None, :]
    rank_col = (float(N - 1) -
                jnp.sum(cmp, axis=1))[:, None]

    rr = lax.broadcasted_iota(jnp.int32, (N, N), 1).astype(_F)
    pp = lax.broadcasted_iota(jnp.int32, (N, N), 0).astype(_F)
    sel = (jnp.broadcast_to(rank_col, (N, N)) == rr)
    sortf = jnp.sum(jnp.where(sel, pp, 0.0), axis=0)[None, :]

    xi = lax.broadcasted_iota(jnp.int32, (1, 64), 1) % 8
    rows = []
    for m in range(3):
        a = sortf[:, m * 64:(m + 1) * 64]
        ap = jnp.pad(a, ((0, 0), (16, 16)))
        for dy in range(3):
            for dx in range(3):
                delta = 8 * (dy - 1) + (dx - 1)
                sh = ap[:, 16 + delta:80 + delta]
                if dx == 0:
                    sh = jnp.where(xi == 0, 0.0, sh)
                elif dx == 2:
                    sh = jnp.where(xi == 7, 0.0, sh)
                rows.append(sh)
    patches = jnp.concatenate(rows, axis=0)
    f24 = jnp.maximum(_dotbf(w1_ref[...], patches) + cb_ref[...], 0.0)
    feats = jnp.concatenate([f24[8 * m:8 * m + 8] for m in range(3)],
                            axis=1)

    o2 = (jnp.broadcast_to(rank_row, (N, N)) ==
          lax.broadcasted_iota(jnp.int32, (N, N), 0).astype(_F))
    mem = _dotbf(feats, o2.astype(_F))

    memp = jnp.pad(mem, ((0, 0), (16, 16)))
    xi2 = lax.broadcasted_iota(jnp.int32, (1, N), 1) % 8
    rows2 = []
    for dy in range(3):
        for dx in range(3):
            delta = 8 * (dy - 1) + (dx - 1)
            sh = memp[:, 16 + delta:208 + delta]
            if dx == 0:
                sh = jnp.where(xi2 == 0, 0.0, sh)
            elif dx == 2:
                sh = jnp.where(xi2 == 7, 0.0, sh)
            rows2.append(sh)
    patches2 = jnp.concatenate(rows2, axis=0)
    mc = jnp.maximum(_dotbf(w2_ref[...], patches2) + mb_ref[...], 0.0)

    pooled = lax.dot_general(mc, pool_ref[...], (((1,), (0,)), ((), ())),
                             precision=_H) * (1.0 / 12.0)
    t = _bf(pwt_ref[...]) * _bf(pooled)[None, :, :]
    ol = jnp.sum(jnp.sum(t, axis=2), axis=1)[None, :] + pb_ref[...]

    cmp2 = _cmp_matrix(ol, NOPS)
    orank_col = (float(NOPS - 1) - jnp.sum(cmp2, axis=1))[:, None]
    rr2 = lax.broadcasted_iota(jnp.int32, (NOPS, NOPS), 1).astype(_F)
    sel2 = (jnp.broadcast_to(orank_col, (NOPS, NOPS)) == rr2)

    def permute(vals_row):
        vcol = _colb1(vals_row, NOPS)
        return jnp.sum(jnp.where(sel2, vcol, 0.0), axis=0)

    s0 = permute(sortf[:, 0:64])
    s1 = permute(sortf[:, 64:128])
    d = permute(sortf[:, 128:192])

    intra = jnp.concatenate([s1 - s0, d - s1])
    inter = s0[1:] - d[:-1]
    return _penalty(inter), _penalty(intra)


@jax.jit
def kernel(mem_logits_batch, conv_w, conv_b, mem_conv_w, mem_conv_b,
           proj_w, proj_b):
    B = mem_logits_batch.shape[0]
    w1 = jnp.zeros((24, 27), _F)
    cw = conv_w[:, :, 0]
    for m in range(3):
        w1 = w1.at[8 * m:8 * m + 8, 9 * m:9 * m + 9].set(
            cw[m].reshape(8, 9))
    cb24 = conv_b.reshape(24, 1)
    w2 = jnp.transpose(mem_conv_w, (2, 3, 1, 0)).reshape(72, 16).T
    mb_t = mem_conv_b.reshape(16, 1)
    pwt = proj_w.reshape(64, 16, 16)
    pb2 = proj_b.reshape(1, 64)
    pool = jnp.asarray(_P)

    specs = [
        pl.BlockSpec((SPB, 1, N), lambda i: (i, 0, 0)),
        pl.BlockSpec((24, 27), lambda i: (0, 0)),
        pl.BlockSpec((24, 1), lambda i: (0, 0)),
        pl.BlockSpec((16, 72), lambda i: (0, 0)),
        pl.BlockSpec((16, 1), lambda i: (0, 0)),
        pl.BlockSpec((64, 16, 16), lambda i: (0, 0, 0)),
        pl.BlockSpec((1, 64), lambda i: (0, 0)),
        pl.BlockSpec((N, 16), lambda i: (0, 0)),
    ]
    out_specs = [pl.BlockSpec((SPB, 1, 128), lambda i: (i, 0, 0)),
                 pl.BlockSpec((SPB, 1, 128), lambda i: (i, 0, 0))]
    out_shape = [jax.ShapeDtypeStruct((B, 1, 128), _F),
                 jax.ShapeDtypeStruct((B, 1, 128), _F)]
    inter, intra = pl.pallas_call(
        _body,
        grid=(B // SPB,),
        in_specs=specs,
        out_specs=out_specs,
        out_shape=out_shape,
    )(mem_logits_batch.reshape(B, 1, N), w1, cb24, w2, mb_t, pwt, pb2, pool)
    return inter[:, 0, 0], intra[:, 0, 0]


# speedup vs baseline: 1.7512x; 1.0073x over previous
import jax
import jax.numpy as jnp
import numpy as np
from jax import lax
from jax.experimental import pallas as pl

_H = jax.lax.Precision.HIGHEST
_BF = jnp.bfloat16
_F = jnp.float32

N = 192
NOPS = 64
WIDTH = 8
ROWS = 24

_P = np.zeros((N, 16), dtype=np.float32)
for _y in range(ROWS):
    for _x in range(WIDTH):
        _P[_y * WIDTH + _x, (_y // 6) * 4 + (_x // 2)] = 1.0


def _bf(v):
    return v.astype(_BF).astype(_F)


def _dotbf(a, b):
    return lax.dot_general(a.astype(_BF), b.astype(_BF),
                           (((1,), (0,)), ((), ())),
                           preferred_element_type=_F)


def _colb3(v_row, n):
    hi = _bf(v_row)
    lo = _bf(v_row - hi)
    rest = v_row - hi - lo
    ones = jnp.ones((1, n), _BF)

    def outer(part):
        return lax.dot_general(part.astype(_BF), ones,
                               (((0,), (0,)), ((), ())),
                               preferred_element_type=_F)
    return outer(hi) + outer(lo) + outer(rest)


def _colb1(v_row, n):
    ones = jnp.ones((1, n), _BF)
    return lax.dot_general(v_row.astype(_BF), ones,
                           (((0,), (0,)), ((), ())),
                           preferred_element_type=_F)


def _cmp_matrix(v_row, n):
    vcol = _colb3(v_row, n)
    vrow = jnp.broadcast_to(v_row, (n, n))
    jj = lax.broadcasted_iota(jnp.int32, (n, n), 0)
    ii = lax.broadcasted_iota(jnp.int32, (n, n), 1)
    cmp = (vcol < vrow) | ((vcol == vrow) & (jj < ii))
    return cmp.astype(_F)


def _staged(h):
    return jnp.where(h <= 2.0, 1.0,
           jnp.where(h <= 4.0, 1.5,
           jnp.where(h <= 8.0, 2.0,
           jnp.where(h <= 16.0, 3.0, 5.0))))


def _penalty(x):
    f = jnp.maximum(x, 0.0)
    b = jnp.maximum(-x, 0.0)
    return jnp.sum(f * _staged(f) + b * b * _staged(b))


SPB = 8


def _body(x_ref, w1_ref, cb_ref, w2_ref, mb_ref, pwt_ref, pb_ref, pool_ref,
          inter_ref, intra_ref):
    for s in range(SPB):
        ip, xp = _sample(x_ref[s], w1_ref, cb_ref, w2_ref, mb_ref, pwt_ref,
                         pb_ref, pool_ref)
        inter_ref[s, 0, :] = jnp.zeros((128,), _F) + ip
        intra_ref[s, 0, :] = jnp.zeros((128,), _F) + xp


def _sample(x, w1_ref, cb_ref, w2_ref, mb_ref, pwt_ref, pb_ref, pool_ref):
    cmp = _cmp_matrix(x, N)
    rank_row = jnp.sum(cmp, axis=0)[None, :]
    rank_col = (float(N - 1) -
                jnp.sum(cmp, axis=1))[:, None]

    rr = lax.broadcasted_iota(jnp.int32, (N, N), 1).astype(_F)
    pp = lax.broadcasted_iota(jnp.int32, (N, N), 0).astype(_F)
    sel = (jnp.broadcast_to(rank_col, (N, N)) == rr)
    sortf = jnp.sum(jnp.where(sel, pp, 0.0), axis=0)[None, :]

    xi = lax.broadcasted_iota(jnp.int32, (1, 64), 1) % 8
    rows = []
    for m in range(3):
        a = sortf[:, m * 64:(m + 1) * 64]
        ap = jnp.pad(a, ((0, 0), (16, 16)))
        for dy in range(3):
            for dx in range(3):
                delta = 8 * (dy - 1) + (dx - 1)
                sh = ap[:, 16 + delta:80 + delta]
                if dx == 0:
                    sh = jnp.where(xi == 0, 0.0, sh)
                elif dx == 2:
                    sh = jnp.where(xi == 7, 0.0, sh)
                rows.append(sh)
    patches = jnp.concatenate(rows, axis=0)
    f24 = jnp.maximum(_dotbf(w1_ref[...], patches) + cb_ref[...], 0.0)
    feats = jnp.concatenate([f24[8 * m:8 * m + 8] for m in range(3)],
                            axis=1)

    o2 = (jnp.broadcast_to(rank_row, (N, N)) ==
          lax.broadcasted_iota(jnp.int32, (N, N), 0).astype(_F))
    mem = _dotbf(feats, o2.astype(_F))

    memp = jnp.pad(mem, ((0, 0), (16, 16)))
    xi2 = lax.broadcasted_iota(jnp.int32, (1, N), 1) % 8
    rows2 = []
    for dy in range(3):
        for dx in range(3):
            delta = 8 * (dy - 1) + (dx - 1)
            sh = memp[:, 16 + delta:208 + delta]
            if dx == 0:
                sh = jnp.where(xi2 == 0, 0.0, sh)
            elif dx == 2:
                sh = jnp.where(xi2 == 7, 0.0, sh)
            rows2.append(sh)
    patches2 = jnp.concatenate(rows2, axis=0)
    mc = jnp.maximum(_dotbf(w2_ref[...], patches2) + mb_ref[...], 0.0)

    pooled = lax.dot_general(mc, pool_ref[...], (((1,), (0,)), ((), ())),
                             precision=_H) * (1.0 / 12.0)
    t = _bf(pwt_ref[...]) * _bf(pooled)[None, :, :]
    ol = jnp.sum(jnp.sum(t, axis=2), axis=1)[None, :] + pb_ref[...]

    cmp2 = _cmp_matrix(ol, NOPS)
    orank_col = (float(NOPS - 1) - jnp.sum(cmp2, axis=1))[:, None]
    rr2 = lax.broadcasted_iota(jnp.int32, (NOPS, NOPS), 1).astype(_F)
    sel2 = (jnp.broadcast_to(orank_col, (NOPS, NOPS)) == rr2)

    def permute(vals_row):
        vcol = _colb1(vals_row, NOPS)
        return jnp.sum(jnp.where(sel2, vcol, 0.0), axis=0)

    s0 = permute(sortf[:, 0:64])
    s1 = permute(sortf[:, 64:128])
    d = permute(sortf[:, 128:192])

    intra = jnp.concatenate([s1 - s0, d - s1])
    inter = s0[1:] - d[:-1]
    return _penalty(inter), _penalty(intra)


@jax.jit
def kernel(mem_logits_batch, conv_w, conv_b, mem_conv_w, mem_conv_b,
           proj_w, proj_b):
    B = mem_logits_batch.shape[0]
    w1 = jnp.zeros((24, 27), _F)
    cw = conv_w[:, :, 0]
    for m in range(3):
        w1 = w1.at[8 * m:8 * m + 8, 9 * m:9 * m + 9].set(
            cw[m].reshape(8, 9))
    cb24 = conv_b.reshape(24, 1)
    w2 = jnp.transpose(mem_conv_w, (2, 3, 1, 0)).reshape(72, 16).T
    mb_t = mem_conv_b.reshape(16, 1)
    pwt = proj_w.reshape(64, 16, 16)
    pb2 = proj_b.reshape(1, 64)
    pool = jnp.asarray(_P)

    specs = [
        pl.BlockSpec((SPB, 1, N), lambda i: (i, 0, 0)),
        pl.BlockSpec((24, 27), lambda i: (0, 0)),
        pl.BlockSpec((24, 1), lambda i: (0, 0)),
        pl.BlockSpec((16, 72), lambda i: (0, 0)),
        pl.BlockSpec((16, 1), lambda i: (0, 0)),
        pl.BlockSpec((64, 16, 16), lambda i: (0, 0, 0)),
        pl.BlockSpec((1, 64), lambda i: (0, 0)),
        pl.BlockSpec((N, 16), lambda i: (0, 0)),
    ]
    out_specs = [pl.BlockSpec((SPB, 1, 128), lambda i: (i, 0, 0)),
                 pl.BlockSpec((SPB, 1, 128), lambda i: (i, 0, 0))]
    out_shape = [jax.ShapeDtypeStruct((B, 1, 128), _F),
                 jax.ShapeDtypeStruct((B, 1, 128), _F)]
    inter, intra = pl.pallas_call(
        _body,
        grid=(B // SPB,),
        in_specs=specs,
        out_specs=out_specs,
        out_shape=out_shape,
    )(mem_logits_batch.reshape(B, 1, N), w1, cb24, w2, mb_t, pwt, pb2, pool)
    return inter[:, 0, 0], intra[:, 0, 0]


# speedup vs baseline: 1.7677x; 1.0094x over previous
import jax
import jax.numpy as jnp
import numpy as np
from jax import lax
from jax.experimental import pallas as pl

_H = jax.lax.Precision.HIGHEST
_BF = jnp.bfloat16
_F = jnp.float32

N = 192
NOPS = 64
WIDTH = 8
ROWS = 24

_P = np.zeros((N, 16), dtype=np.float32)
for _y in range(ROWS):
    for _x in range(WIDTH):
        _P[_y * WIDTH + _x, (_y // 6) * 4 + (_x // 2)] = 1.0


def _bf(v):
    return v.astype(_BF).astype(_F)


def _dotbf(a, b):
    return lax.dot_general(a.astype(_BF), b.astype(_BF),
                           (((1,), (0,)), ((), ())),
                           preferred_element_type=_F)


def _colb3(v_row, n):
    hi = _bf(v_row)
    lo = _bf(v_row - hi)
    rest = v_row - hi - lo
    ones = jnp.ones((1, n), _BF)

    def outer(part):
        return lax.dot_general(part.astype(_BF), ones,
                               (((0,), (0,)), ((), ())),
                               preferred_element_type=_F)
    return outer(hi) + outer(lo) + outer(rest)


def _colb1(v_row, n):
    ones = jnp.ones((1, n), _BF)
    return lax.dot_general(v_row.astype(_BF), ones,
                           (((0,), (0,)), ((), ())),
                           preferred_element_type=_F)


def _cmp_matrix(v_row, n):
    vcol = _colb3(v_row, n)
    vrow = jnp.broadcast_to(v_row, (n, n))
    jj = lax.broadcasted_iota(jnp.int32, (n, n), 0)
    ii = lax.broadcasted_iota(jnp.int32, (n, n), 1)
    cmp = (vcol < vrow) | ((vcol == vrow) & (jj < ii))
    return cmp.astype(_F)


def _staged(h):
    return jnp.where(h <= 2.0, 1.0,
           jnp.where(h <= 4.0, 1.5,
           jnp.where(h <= 8.0, 2.0,
           jnp.where(h <= 16.0, 3.0, 5.0))))


def _penalty(x):
    f = jnp.maximum(x, 0.0)
    b = jnp.maximum(-x, 0.0)
    return jnp.sum(f * _staged(f) + b * b * _staged(b))


SPB = 16


def _body(x_ref, w1_ref, cb_ref, w2_ref, mb_ref, pwt_ref, pb_ref, pool_ref,
          inter_ref, intra_ref):
    for s in range(SPB):
        ip, xp = _sample(x_ref[s], w1_ref, cb_ref, w2_ref, mb_ref, pwt_ref,
                         pb_ref, pool_ref)
        inter_ref[s, 0, :] = jnp.zeros((128,), _F) + ip
        intra_ref[s, 0, :] = jnp.zeros((128,), _F) + xp


def _sample(x, w1_ref, cb_ref, w2_ref, mb_ref, pwt_ref, pb_ref, pool_ref):
    cmp = _cmp_matrix(x, N)
    rank_row = jnp.sum(cmp, axis=0)[None, :]
    rank_col = (float(N - 1) -
                jnp.sum(cmp, axis=1))[:, None]

    rr = lax.broadcasted_iota(jnp.int32, (N, N), 1).astype(_F)
    pp = lax.broadcasted_iota(jnp.int32, (N, N), 0).astype(_F)
    sel = (jnp.broadcast_to(rank_col, (N, N)) == rr)
    sortf = jnp.sum(jnp.where(sel, pp, 0.0), axis=0)[None, :]

    xi = lax.broadcasted_iota(jnp.int32, (1, 64), 1) % 8
    rows = []
    for m in range(3):
        a = sortf[:, m * 64:(m + 1) * 64]
        ap = jnp.pad(a, ((0, 0), (16, 16)))
        for dy in range(3):
            for dx in range(3):
                delta = 8 * (dy - 1) + (dx - 1)
                sh = ap[:, 16 + delta:80 + delta]
                if dx == 0:
                    sh = jnp.where(xi == 0, 0.0, sh)
                elif dx == 2:
                    sh = jnp.where(xi == 7, 0.0, sh)
                rows.append(sh)
    patches = jnp.concatenate(rows, axis=0)
    f24 = jnp.maximum(_dotbf(w1_ref[...], patches) + cb_ref[...], 0.0)
    feats = jnp.concatenate([f24[8 * m:8 * m + 8] for m in range(3)],
                            axis=1)

    o2 = (jnp.broadcast_to(rank_row, (N, N)) ==
          lax.broadcasted_iota(jnp.int32, (N, N), 0).astype(_F))
    mem = _dotbf(feats, o2.astype(_F))

    memp = jnp.pad(mem, ((0, 0), (16, 16)))
    xi2 = lax.broadcasted_iota(jnp.int32, (1, N), 1) % 8
    rows2 = []
    for dy in range(3):
        for dx in range(3):
            delta = 8 * (dy - 1) + (dx - 1)
            sh = memp[:, 16 + delta:208 + delta]
            if dx == 0:
                sh = jnp.where(xi2 == 0, 0.0, sh)
            elif dx == 2:
                sh = jnp.where(xi2 == 7, 0.0, sh)
            rows2.append(sh)
    patches2 = jnp.concatenate(rows2, axis=0)
    mc = jnp.maximum(_dotbf(w2_ref[...], patches2) + mb_ref[...], 0.0)

    pooled = lax.dot_general(mc, pool_ref[...], (((1,), (0,)), ((), ())),
                             precision=_H) * (1.0 / 12.0)
    t = _bf(pwt_ref[...]) * _bf(pooled)[None, :, :]
    ol = jnp.sum(jnp.sum(t, axis=2), axis=1)[None, :] + pb_ref[...]

    cmp2 = _cmp_matrix(ol, NOPS)
    orank_col = (float(NOPS - 1) - jnp.sum(cmp2, axis=1))[:, None]
    rr2 = lax.broadcasted_iota(jnp.int32, (NOPS, NOPS), 1).astype(_F)
    sel2 = (jnp.broadcast_to(orank_col, (NOPS, NOPS)) == rr2)

    def permute(vals_row):
        vcol = _colb1(vals_row, NOPS)
        return jnp.sum(jnp.where(sel2, vcol, 0.0), axis=0)

    s0 = permute(sortf[:, 0:64])
    s1 = permute(sortf[:, 64:128])
    d = permute(sortf[:, 128:192])

    intra = jnp.concatenate([s1 - s0, d - s1])
    inter = s0[1:] - d[:-1]
    return _penalty(inter), _penalty(intra)


@jax.jit
def kernel(mem_logits_batch, conv_w, conv_b, mem_conv_w, mem_conv_b,
           proj_w, proj_b):
    B = mem_logits_batch.shape[0]
    w1 = jnp.zeros((24, 27), _F)
    cw = conv_w[:, :, 0]
    for m in range(3):
        w1 = w1.at[8 * m:8 * m + 8, 9 * m:9 * m + 9].set(
            cw[m].reshape(8, 9))
    cb24 = conv_b.reshape(24, 1)
    w2 = jnp.transpose(mem_conv_w, (2, 3, 1, 0)).reshape(72, 16).T
    mb_t = mem_conv_b.reshape(16, 1)
    pwt = proj_w.reshape(64, 16, 16)
    pb2 = proj_b.reshape(1, 64)
    pool = jnp.asarray(_P)

    specs = [
        pl.BlockSpec((SPB, 1, N), lambda i: (i, 0, 0)),
        pl.BlockSpec((24, 27), lambda i: (0, 0)),
        pl.BlockSpec((24, 1), lambda i: (0, 0)),
        pl.BlockSpec((16, 72), lambda i: (0, 0)),
        pl.BlockSpec((16, 1), lambda i: (0, 0)),
        pl.BlockSpec((64, 16, 16), lambda i: (0, 0, 0)),
        pl.BlockSpec((1, 64), lambda i: (0, 0)),
        pl.BlockSpec((N, 16), lambda i: (0, 0)),
    ]
    out_specs = [pl.BlockSpec((SPB, 1, 128), lambda i: (i, 0, 0)),
                 pl.BlockSpec((SPB, 1, 128), lambda i: (i, 0, 0))]
    out_shape = [jax.ShapeDtypeStruct((B, 1, 128), _F),
                 jax.ShapeDtypeStruct((B, 1, 128), _F)]
    inter, intra = pl.pallas_call(
        _body,
        grid=(B // SPB,),
        in_specs=specs,
        out_specs=out_specs,
        out_shape=out_shape,
    )(mem_logits_batch.reshape(B, 1, N), w1, cb24, w2, mb_t, pwt, pb2, pool)
    return inter[:, 0, 0], intra[:, 0, 0]


# speedup vs baseline: 2.9820x; 1.6870x over previous
import jax
import jax.numpy as jnp
import numpy as np
from jax import lax
from jax.experimental import pallas as pl

_H = jax.lax.Precision.HIGHEST
_BF = jnp.bfloat16
_F = jnp.float32

N = 192
NOPS = 64
WIDTH = 8
ROWS = 24

_P = np.zeros((N, 16), dtype=np.float32)
for _y in range(ROWS):
    for _x in range(WIDTH):
        _P[_y * WIDTH + _x, (_y // 6) * 4 + (_x // 2)] = 1.0


def _bf(v):
    return v.astype(_BF).astype(_F)


def _dotbf(a, b):
    return lax.dot_general(a.astype(_BF), b.astype(_BF),
                           (((1,), (0,)), ((), ())),
                           preferred_element_type=_F)


def _colb3(v_row, n):
    hi = _bf(v_row)
    lo = _bf(v_row - hi)
    rest = v_row - hi - lo
    ones = jnp.ones((1, n), _BF)

    def outer(part):
        return lax.dot_general(part.astype(_BF), ones,
                               (((0,), (0,)), ((), ())),
                               preferred_element_type=_F)
    return outer(hi) + outer(lo) + outer(rest)


def _onehot_bf(cond):
    return jnp.where(cond, 1.0, 0.0).astype(_BF)


def _dot_raw(a_bf, b_bf):
    return lax.dot_general(a_bf, b_bf, (((1,), (0,)), ((), ())),
                           preferred_element_type=_F)


def _colb1(v_row, n):
    ones = jnp.ones((1, n), _BF)
    return lax.dot_general(v_row.astype(_BF), ones,
                           (((0,), (0,)), ((), ())),
                           preferred_element_type=_F)


def _cmp_matrix(v_row, n):
    vcol = _colb3(v_row, n)
    vrow = jnp.broadcast_to(v_row, (n, n))
    jj = lax.broadcasted_iota(jnp.int32, (n, n), 0)
    ii = lax.broadcasted_iota(jnp.int32, (n, n), 1)
    cmp = (vcol < vrow) | ((vcol == vrow) & (jj < ii))
    return cmp.astype(_F)


def _staged(h):
    return jnp.where(h <= 2.0, 1.0,
           jnp.where(h <= 4.0, 1.5,
           jnp.where(h <= 8.0, 2.0,
           jnp.where(h <= 16.0, 3.0, 5.0))))


def _penalty(x):
    f = jnp.maximum(x, 0.0)
    b = jnp.maximum(-x, 0.0)
    return jnp.sum(f * _staged(f) + b * b * _staged(b))


SPB = 16


def _body(x_ref, w1_ref, cb_ref, w2_ref, mb_ref, pwt_ref, pb_ref, pool_ref,
          inter_ref, intra_ref):
    S = SPB
    jj = lax.broadcasted_iota(jnp.int32, (N, N), 0)
    ii = lax.broadcasted_iota(jnp.int32, (N, N), 1)
    jjf = jj.astype(_F)
    rrf = ii.astype(_F)
    ones_r = jnp.ones((1, N), _BF)
    ones_c = jnp.ones((N, 1), _BF)
    iota_bf = rrf[0:1, :].astype(_BF)
    xi = lax.broadcasted_iota(jnp.int32, (1, 64), 1) % 8
    xi2 = lax.broadcasted_iota(jnp.int32, (1, N), 1) % 8
    rr2f = lax.broadcasted_iota(jnp.int32, (NOPS, NOPS), 1).astype(_F)
    jj2 = lax.broadcasted_iota(jnp.int32, (NOPS, NOPS), 0)
    ii2 = lax.broadcasted_iota(jnp.int32, (NOPS, NOPS), 1)

    cmpb = []
    for s in range(S):
        x = x_ref[s]
        vcol = _colb3(x, N)
        vrow = jnp.broadcast_to(x, (N, N))
        cond = (vcol < vrow) | ((vcol == vrow) & (jj < ii))
        cmpb.append(_onehot_bf(cond))

    rank_row = [_dot_raw(ones_r, cmpb[s]) for s in range(S)]
    rank_col = [float(N - 1) - _dot_raw(cmpb[s], ones_c)
                for s in range(S)]

    sortf = []
    for s in range(S):
        ob = _onehot_bf(jnp.broadcast_to(rank_col[s], (N, N)) == rrf)
        sortf.append(_dot_raw(iota_bf, ob))

    feats = []
    for s in range(S):
        rows = []
        for m in range(3):
            ap = jnp.pad(sortf[s][:, m * 64:(m + 1) * 64],
                         ((0, 0), (16, 16)))
            for dy in range(3):
                for dx in range(3):
                    delta = 8 * (dy - 1) + (dx - 1)
                    sh = ap[:, 16 + delta:80 + delta]
                    if dx == 0:
                        sh = jnp.where(xi == 0, 0.0, sh)
                    elif dx == 2:
                        sh = jnp.where(xi == 7, 0.0, sh)
                    rows.append(sh)
        patches = jnp.concatenate(rows, axis=0)
        f24 = jnp.maximum(_dotbf(w1_ref[...], patches) + cb_ref[...], 0.0)
        feats.append(jnp.concatenate(
            [f24[8 * m:8 * m + 8] for m in range(3)], axis=1))

    mem = []
    for s in range(S):
        o2 = _onehot_bf(jnp.broadcast_to(rank_row[s], (N, N)) == jjf)
        mem.append(_dot_raw(feats[s].astype(_BF), o2))

    mcs = []
    for s in range(S):
        memp = jnp.pad(mem[s], ((0, 0), (16, 16)))
        rows2 = []
        for dy in range(3):
            for dx in range(3):
                delta = 8 * (dy - 1) + (dx - 1)
                sh = memp[:, 16 + delta:208 + delta]
                if dx == 0:
                    sh = jnp.where(xi2 == 0, 0.0, sh)
                elif dx == 2:
                    sh = jnp.where(xi2 == 7, 0.0, sh)
                rows2.append(sh)
        patches2 = jnp.concatenate(rows2, axis=0)
        mcs.append(jnp.maximum(
            _dotbf(w2_ref[...], patches2) + mb_ref[...], 0.0))

    ols = []
    for s in range(S):
        pooled = lax.dot_general(mcs[s], pool_ref[...],
                                 (((1,), (0,)), ((), ())),
                                 precision=_H) * (1.0 / 12.0)
        t = _bf(pwt_ref[...]) * _bf(pooled)[None, :, :]
        ols.append(jnp.sum(jnp.sum(t, axis=2), axis=1)[None, :]
                   + pb_ref[...])

    ones2_c = jnp.ones((NOPS, 1), _BF)
    for s in range(S):
        ol = ols[s]
        vcol2 = _colb3(ol, NOPS)
        vrow2 = jnp.broadcast_to(ol, (NOPS, NOPS))
        cond2 = (vcol2 < vrow2) | ((vcol2 == vrow2) & (jj2 < ii2))
        c2b = _onehot_bf(cond2)
        orank_col = float(NOPS - 1) - _dot_raw(c2b, ones2_c)
        o3 = _onehot_bf(jnp.broadcast_to(orank_col, (NOPS, NOPS)) == rr2f)
        s0 = _dot_raw(sortf[s][:, 0:64].astype(_BF), o3)[0]
        s1 = _dot_raw(sortf[s][:, 64:128].astype(_BF), o3)[0]
        d = _dot_raw(sortf[s][:, 128:192].astype(_BF), o3)[0]
        intra = jnp.concatenate([s1 - s0, d - s1])
        inter = s0[1:] - d[:-1]
        inter_ref[s, 0, :] = jnp.zeros((128,), _F) + _penalty(inter)
        intra_ref[s, 0, :] = jnp.zeros((128,), _F) + _penalty(intra)


@jax.jit
def kernel(mem_logits_batch, conv_w, conv_b, mem_conv_w, mem_conv_b,
           proj_w, proj_b):
    B = mem_logits_batch.shape[0]
    w1 = jnp.zeros((24, 27), _F)
    cw = conv_w[:, :, 0]
    for m in range(3):
        w1 = w1.at[8 * m:8 * m + 8, 9 * m:9 * m + 9].set(
            cw[m].reshape(8, 9))
    cb24 = conv_b.reshape(24, 1)
    w2 = jnp.transpose(mem_conv_w, (2, 3, 1, 0)).reshape(72, 16).T
    mb_t = mem_conv_b.reshape(16, 1)
    pwt = proj_w.reshape(64, 16, 16)
    pb2 = proj_b.reshape(1, 64)
    pool = jnp.asarray(_P)

    specs = [
        pl.BlockSpec((SPB, 1, N), lambda i: (i, 0, 0)),
        pl.BlockSpec((24, 27), lambda i: (0, 0)),
        pl.BlockSpec((24, 1), lambda i: (0, 0)),
        pl.BlockSpec((16, 72), lambda i: (0, 0)),
        pl.BlockSpec((16, 1), lambda i: (0, 0)),
        pl.BlockSpec((64, 16, 16), lambda i: (0, 0, 0)),
        pl.BlockSpec((1, 64), lambda i: (0, 0)),
        pl.BlockSpec((N, 16), lambda i: (0, 0)),
    ]
    out_specs = [pl.BlockSpec((SPB, 1, 128), lambda i: (i, 0, 0)),
                 pl.BlockSpec((SPB, 1, 128), lambda i: (i, 0, 0))]
    out_shape = [jax.ShapeDtypeStruct((B, 1, 128), _F),
                 jax.ShapeDtypeStruct((B, 1, 128), _F)]
    inter, intra = pl.pallas_call(
        _body,
        grid=(B // SPB,),
        in_specs=specs,
        out_specs=out_specs,
        out_shape=out_shape,
    )(mem_logits_batch.reshape(B, 1, N), w1, cb24, w2, mb_t, pwt, pb2, pool)
    return inter[:, 0, 0], intra[:, 0, 0]


# speedup vs baseline: 2.9946x; 1.0042x over previous
import jax
import jax.numpy as jnp
import numpy as np
from jax import lax
from jax.experimental import pallas as pl

_H = jax.lax.Precision.HIGHEST
_BF = jnp.bfloat16
_F = jnp.float32

N = 192
NOPS = 64
WIDTH = 8
ROWS = 24

_P = np.zeros((N, 16), dtype=np.float32)
for _y in range(ROWS):
    for _x in range(WIDTH):
        _P[_y * WIDTH + _x, (_y // 6) * 4 + (_x // 2)] = 1.0


def _bf(v):
    return v.astype(_BF).astype(_F)


def _dotbf(a, b):
    return lax.dot_general(a.astype(_BF), b.astype(_BF),
                           (((1,), (0,)), ((), ())),
                           preferred_element_type=_F)


def _colb3(v_row, n):
    hi = _bf(v_row)
    lo = _bf(v_row - hi)
    rest = v_row - hi - lo
    ones = jnp.ones((1, n), _BF)

    def outer(part):
        return lax.dot_general(part.astype(_BF), ones,
                               (((0,), (0,)), ((), ())),
                               preferred_element_type=_F)
    return outer(hi) + outer(lo) + outer(rest)


def _onehot_bf(cond):
    return jnp.where(cond, 1.0, 0.0).astype(_BF)


def _dot_raw(a_bf, b_bf):
    return lax.dot_general(a_bf, b_bf, (((1,), (0,)), ((), ())),
                           preferred_element_type=_F)


def _colb1(v_row, n):
    ones = jnp.ones((1, n), _BF)
    return lax.dot_general(v_row.astype(_BF), ones,
                           (((0,), (0,)), ((), ())),
                           preferred_element_type=_F)


def _cmp_matrix(v_row, n):
    vcol = _colb3(v_row, n)
    vrow = jnp.broadcast_to(v_row, (n, n))
    jj = lax.broadcasted_iota(jnp.int32, (n, n), 0)
    ii = lax.broadcasted_iota(jnp.int32, (n, n), 1)
    cmp = (vcol < vrow) | ((vcol == vrow) & (jj < ii))
    return cmp.astype(_F)


def _staged(h):
    return jnp.where(h <= 2.0, 1.0,
           jnp.where(h <= 4.0, 1.5,
           jnp.where(h <= 8.0, 2.0,
           jnp.where(h <= 16.0, 3.0, 5.0))))


def _penalty(x):
    f = jnp.maximum(x, 0.0)
    b = jnp.maximum(-x, 0.0)
    return jnp.sum(f * _staged(f) + b * b * _staged(b))


SPB = 32


def _body(x_ref, w1_ref, cb_ref, w2_ref, mb_ref, pwt_ref, pb_ref, pool_ref,
          inter_ref, intra_ref):
    S = SPB
    jj = lax.broadcasted_iota(jnp.int32, (N, N), 0)
    ii = lax.broadcasted_iota(jnp.int32, (N, N), 1)
    jjf = jj.astype(_F)
    rrf = ii.astype(_F)
    ones_r = jnp.ones((1, N), _BF)
    ones_c = jnp.ones((N, 1), _BF)
    iota_bf = rrf[0:1, :].astype(_BF)
    xi = lax.broadcasted_iota(jnp.int32, (1, 64), 1) % 8
    xi2 = lax.broadcasted_iota(jnp.int32, (1, N), 1) % 8
    rr2f = lax.broadcasted_iota(jnp.int32, (NOPS, NOPS), 1).astype(_F)
    jj2 = lax.broadcasted_iota(jnp.int32, (NOPS, NOPS), 0)
    ii2 = lax.broadcasted_iota(jnp.int32, (NOPS, NOPS), 1)

    cmpb = []
    for s in range(S):
        x = x_ref[s]
        vcol = _colb3(x, N)
        vrow = jnp.broadcast_to(x, (N, N))
        cond = (vcol < vrow) | ((vcol == vrow) & (jj < ii))
        cmpb.append(_onehot_bf(cond))

    rank_row = [_dot_raw(ones_r, cmpb[s]) for s in range(S)]
    rank_col = [float(N - 1) - _dot_raw(cmpb[s], ones_c)
                for s in range(S)]

    sortf = []
    for s in range(S):
        ob = _onehot_bf(jnp.broadcast_to(rank_col[s], (N, N)) == rrf)
        sortf.append(_dot_raw(iota_bf, ob))

    feats = []
    for s in range(S):
        rows = []
        for m in range(3):
            ap = jnp.pad(sortf[s][:, m * 64:(m + 1) * 64],
                         ((0, 0), (16, 16)))
            for dy in range(3):
                for dx in range(3):
                    delta = 8 * (dy - 1) + (dx - 1)
                    sh = ap[:, 16 + delta:80 + delta]
                    if dx == 0:
                        sh = jnp.where(xi == 0, 0.0, sh)
                    elif dx == 2:
                        sh = jnp.where(xi == 7, 0.0, sh)
                    rows.append(sh)
        patches = jnp.concatenate(rows, axis=0)
        f24 = jnp.maximum(_dotbf(w1_ref[...], patches) + cb_ref[...], 0.0)
        feats.append(jnp.concatenate(
            [f24[8 * m:8 * m + 8] for m in range(3)], axis=1))

    mem = []
    for s in range(S):
        o2 = _onehot_bf(jnp.broadcast_to(rank_row[s], (N, N)) == jjf)
        mem.append(_dot_raw(feats[s].astype(_BF), o2))

    mcs = []
    for s in range(S):
        memp = jnp.pad(mem[s], ((0, 0), (16, 16)))
        rows2 = []
        for dy in range(3):
            for dx in range(3):
                delta = 8 * (dy - 1) + (dx - 1)
                sh = memp[:, 16 + delta:208 + delta]
                if dx == 0:
                    sh = jnp.where(xi2 == 0, 0.0, sh)
                elif dx == 2:
                    sh = jnp.where(xi2 == 7, 0.0, sh)
                rows2.append(sh)
        patches2 = jnp.concatenate(rows2, axis=0)
        mcs.append(jnp.maximum(
            _dotbf(w2_ref[...], patches2) + mb_ref[...], 0.0))

    ols = []
    for s in range(S):
        pooled = lax.dot_general(mcs[s], pool_ref[...],
                                 (((1,), (0,)), ((), ())),
                                 precision=_H) * (1.0 / 12.0)
        t = _bf(pwt_ref[...]) * _bf(pooled)[None, :, :]
        ols.append(jnp.sum(jnp.sum(t, axis=2), axis=1)[None, :]
                   + pb_ref[...])

    ones2_c = jnp.ones((NOPS, 1), _BF)
    for s in range(S):
        ol = ols[s]
        vcol2 = _colb3(ol, NOPS)
        vrow2 = jnp.broadcast_to(ol, (NOPS, NOPS))
        cond2 = (vcol2 < vrow2) | ((vcol2 == vrow2) & (jj2 < ii2))
        c2b = _onehot_bf(cond2)
        orank_col = float(NOPS - 1) - _dot_raw(c2b, ones2_c)
        o3 = _onehot_bf(jnp.broadcast_to(orank_col, (NOPS, NOPS)) == rr2f)
        s0 = _dot_raw(sortf[s][:, 0:64].astype(_BF), o3)[0]
        s1 = _dot_raw(sortf[s][:, 64:128].astype(_BF), o3)[0]
        d = _dot_raw(sortf[s][:, 128:192].astype(_BF), o3)[0]
        intra = jnp.concatenate([s1 - s0, d - s1])
        inter = s0[1:] - d[:-1]
        inter_ref[s, 0, :] = jnp.zeros((128,), _F) + _penalty(inter)
        intra_ref[s, 0, :] = jnp.zeros((128,), _F) + _penalty(intra)


@jax.jit
def kernel(mem_logits_batch, conv_w, conv_b, mem_conv_w, mem_conv_b,
           proj_w, proj_b):
    B = mem_logits_batch.shape[0]
    w1 = jnp.zeros((24, 27), _F)
    cw = conv_w[:, :, 0]
    for m in range(3):
        w1 = w1.at[8 * m:8 * m + 8, 9 * m:9 * m + 9].set(
            cw[m].reshape(8, 9))
    cb24 = conv_b.reshape(24, 1)
    w2 = jnp.transpose(mem_conv_w, (2, 3, 1, 0)).reshape(72, 16).T
    mb_t = mem_conv_b.reshape(16, 1)
    pwt = proj_w.reshape(64, 16, 16)
    pb2 = proj_b.reshape(1, 64)
    pool = jnp.asarray(_P)

    specs = [
        pl.BlockSpec((SPB, 1, N), lambda i: (i, 0, 0)),
        pl.BlockSpec((24, 27), lambda i: (0, 0)),
        pl.BlockSpec((24, 1), lambda i: (0, 0)),
        pl.BlockSpec((16, 72), lambda i: (0, 0)),
        pl.BlockSpec((16, 1), lambda i: (0, 0)),
        pl.BlockSpec((64, 16, 16), lambda i: (0, 0, 0)),
        pl.BlockSpec((1, 64), lambda i: (0, 0)),
        pl.BlockSpec((N, 16), lambda i: (0, 0)),
    ]
    out_specs = [pl.BlockSpec((SPB, 1, 128), lambda i: (i, 0, 0)),
                 pl.BlockSpec((SPB, 1, 128), lambda i: (i, 0, 0))]
    out_shape = [jax.ShapeDtypeStruct((B, 1, 128), _F),
                 jax.ShapeDtypeStruct((B, 1, 128), _F)]
    inter, intra = pl.pallas_call(
        _body,
        grid=(B // SPB,),
        in_specs=specs,
        out_specs=out_specs,
        out_shape=out_shape,
    )(mem_logits_batch.reshape(B, 1, N), w1, cb24, w2, mb_t, pwt, pb2, pool)
    return inter[:, 0, 0], intra[:, 0, 0]


# speedup vs baseline: 2.9955x; 1.0003x over previous
import jax
import jax.numpy as jnp
import numpy as np
from jax import lax
from jax.experimental import pallas as pl

_H = jax.lax.Precision.HIGHEST
_BF = jnp.bfloat16
_F = jnp.float32

N = 192
NOPS = 64
WIDTH = 8
ROWS = 24

_P = np.zeros((N, 16), dtype=np.float32)
for _y in range(ROWS):
    for _x in range(WIDTH):
        _P[_y * WIDTH + _x, (_y // 6) * 4 + (_x // 2)] = 1.0


def _bf(v):
    return v.astype(_BF).astype(_F)


def _dotbf(a, b):
    return lax.dot_general(a.astype(_BF), b.astype(_BF),
                           (((1,), (0,)), ((), ())),
                           preferred_element_type=_F)


def _colb3(v_row, n):
    hi = _bf(v_row)
    lo = _bf(v_row - hi)
    rest = v_row - hi - lo
    ones = jnp.ones((1, n), _BF)

    def outer(part):
        return lax.dot_general(part.astype(_BF), ones,
                               (((0,), (0,)), ((), ())),
                               preferred_element_type=_F)
    return outer(hi) + outer(lo) + outer(rest)


def _onehot_bf(cond):
    return jnp.where(cond, 1.0, 0.0).astype(_BF)


def _dot_raw(a_bf, b_bf):
    return lax.dot_general(a_bf, b_bf, (((1,), (0,)), ((), ())),
                           preferred_element_type=_F)


def _staged(h):
    return jnp.where(h <= 2.0, 1.0,
           jnp.where(h <= 4.0, 1.5,
           jnp.where(h <= 8.0, 2.0,
           jnp.where(h <= 16.0, 3.0, 5.0))))


def _penalty(x):
    f = jnp.maximum(x, 0.0)
    b = jnp.maximum(-x, 0.0)
    return jnp.sum(f * _staged(f) + b * b * _staged(b))


SPB = 32


def _body(x_ref, w1_ref, cb_ref, w2_ref, mb_ref, pwt_ref, pb_ref, pool_ref,
          inter_ref, intra_ref):
    S = SPB
    jj = lax.broadcasted_iota(jnp.int32, (N, N), 0)
    ii = lax.broadcasted_iota(jnp.int32, (N, N), 1)
    jjf = jj.astype(_F)
    rrf = ii.astype(_F)
    ones_r = jnp.ones((1, N), _BF)
    ones_c = jnp.ones((N, 1), _BF)
    iota_bf = rrf[0:1, :].astype(_BF)
    xi = lax.broadcasted_iota(jnp.int32, (1, 64), 1) % 8
    xi2 = lax.broadcasted_iota(jnp.int32, (1, N), 1) % 8
    rr2f = lax.broadcasted_iota(jnp.int32, (NOPS, NOPS), 1).astype(_F)
    jj2 = lax.broadcasted_iota(jnp.int32, (NOPS, NOPS), 0)
    ii2 = lax.broadcasted_iota(jnp.int32, (NOPS, NOPS), 1)

    cmpb = []
    for s in range(S):
        x = x_ref[s]
        vcol = _colb3(x, N)
        vrow = jnp.broadcast_to(x, (N, N))
        cond = (vcol < vrow) | ((vcol == vrow) & (jj < ii))
        cmpb.append(_onehot_bf(cond))

    rank_row = [_dot_raw(ones_r, cmpb[s]) for s in range(S)]
    rank_col = [float(N - 1) - _dot_raw(cmpb[s], ones_c)
                for s in range(S)]

    sortf = []
    for s in range(S):
        ob = _onehot_bf(jnp.broadcast_to(rank_col[s], (N, N)) == rrf)
        sortf.append(_dot_raw(iota_bf, ob))

    feats = []
    for s in range(S):
        rows = []
        for m in range(3):
            ap = jnp.pad(sortf[s][:, m * 64:(m + 1) * 64],
                         ((0, 0), (16, 16)))
            for dy in range(3):
                for dx in range(3):
                    delta = 8 * (dy - 1) + (dx - 1)
                    sh = ap[:, 16 + delta:80 + delta]
                    if dx == 0:
                        sh = jnp.where(xi == 0, 0.0, sh)
                    elif dx == 2:
                        sh = jnp.where(xi == 7, 0.0, sh)
                    rows.append(sh)
        patches = jnp.concatenate(rows, axis=0)
        f24 = jnp.maximum(_dotbf(w1_ref[...], patches) + cb_ref[...], 0.0)
        feats.append(jnp.concatenate(
            [f24[8 * m:8 * m + 8] for m in range(3)], axis=1))

    mem = []
    for s in range(S):
        o2 = _onehot_bf(jnp.broadcast_to(rank_row[s], (N, N)) == jjf)
        mem.append(_dot_raw(feats[s].astype(_BF), o2))

    mcs = []
    for s in range(S):
        memp = jnp.pad(mem[s], ((0, 0), (16, 16)))
        rows2 = []
        for dy in range(3):
            for dx in range(3):
                delta = 8 * (dy - 1) + (dx - 1)
                sh = memp[:, 16 + delta:208 + delta]
                if dx == 0:
                    sh = jnp.where(xi2 == 0, 0.0, sh)
                elif dx == 2:
                    sh = jnp.where(xi2 == 7, 0.0, sh)
                rows2.append(sh)
        patches2 = jnp.concatenate(rows2, axis=0)
        mcs.append(jnp.maximum(
            _dotbf(w2_ref[...], patches2) + mb_ref[...], 0.0))

    ols = []
    for s in range(S):
        pooled = lax.dot_general(mcs[s], pool_ref[...],
                                 (((1,), (0,)), ((), ())),
                                 precision=_H) * (1.0 / 12.0)
        t = _bf(pwt_ref[...]) * _bf(pooled)[None, :, :]
        ols.append(jnp.sum(jnp.sum(t, axis=2), axis=1)[None, :]
                   + pb_ref[...])

    ones2_c = jnp.ones((NOPS, 1), _BF)
    for s in range(S):
        ol = ols[s]
        vcol2 = _colb3(ol, NOPS)
        vrow2 = jnp.broadcast_to(ol, (NOPS, NOPS))
        cond2 = (vcol2 < vrow2) | ((vcol2 == vrow2) & (jj2 < ii2))
        c2b = _onehot_bf(cond2)
        orank_col = float(NOPS - 1) - _dot_raw(c2b, ones2_c)
        o3 = _onehot_bf(jnp.broadcast_to(orank_col, (NOPS, NOPS)) == rr2f)
        s0 = _dot_raw(sortf[s][:, 0:64].astype(_BF), o3)[0]
        s1 = _dot_raw(sortf[s][:, 64:128].astype(_BF), o3)[0]
        d = _dot_raw(sortf[s][:, 128:192].astype(_BF), o3)[0]
        intra = jnp.concatenate([s1 - s0, d - s1])
        inter = s0[1:] - d[:-1]
        inter_ref[s, 0, :] = jnp.zeros((128,), _F) + _penalty(inter)
        intra_ref[s, 0, :] = jnp.zeros((128,), _F) + _penalty(intra)


@jax.jit
def kernel(mem_logits_batch, conv_w, conv_b, mem_conv_w, mem_conv_b,
           proj_w, proj_b):
    B = mem_logits_batch.shape[0]
    w1 = jnp.zeros((24, 27), _F)
    cw = conv_w[:, :, 0]
    for m in range(3):
        w1 = w1.at[8 * m:8 * m + 8, 9 * m:9 * m + 9].set(
            cw[m].reshape(8, 9))
    cb24 = conv_b.reshape(24, 1)
    w2 = jnp.transpose(mem_conv_w, (2, 3, 1, 0)).reshape(72, 16).T
    mb_t = mem_conv_b.reshape(16, 1)
    pwt = proj_w.reshape(64, 16, 16)
    pb2 = proj_b.reshape(1, 64)
    pool = jnp.asarray(_P)

    specs = [
        pl.BlockSpec((SPB, 1, N), lambda i: (i, 0, 0)),
        pl.BlockSpec((24, 27), lambda i: (0, 0)),
        pl.BlockSpec((24, 1), lambda i: (0, 0)),
        pl.BlockSpec((16, 72), lambda i: (0, 0)),
        pl.BlockSpec((16, 1), lambda i: (0, 0)),
        pl.BlockSpec((64, 16, 16), lambda i: (0, 0, 0)),
        pl.BlockSpec((1, 64), lambda i: (0, 0)),
        pl.BlockSpec((N, 16), lambda i: (0, 0)),
    ]
    out_specs = [pl.BlockSpec((SPB, 1, 128), lambda i: (i, 0, 0)),
                 pl.BlockSpec((SPB, 1, 128), lambda i: (i, 0, 0))]
    out_shape = [jax.ShapeDtypeStruct((B, 1, 128), _F),
                 jax.ShapeDtypeStruct((B, 1, 128), _F)]
    inter, intra = pl.pallas_call(
        _body,
        grid=(B // SPB,),
        in_specs=specs,
        out_specs=out_specs,
        out_shape=out_shape,
    )(mem_logits_batch.reshape(B, 1, N), w1, cb24, w2, mb_t, pwt, pb2, pool)
    return inter[:, 0, 0], intra[:, 0, 0]
